# Initial kernel scaffold; baseline (speedup 1.0000x reference)
import jax
import jax.numpy as jnp
from jax.experimental import pallas as pl


def kernel(value_list_0, value_list_1, value_list_2, start, duration):
    raise NotImplementedError("write your pallas kernel here")


# speedup vs baseline: 1.4360x; 1.4360x over previous
import functools

import jax
import jax.numpy as jnp
from jax import lax
from jax.experimental import pallas as pl

_RATIOS = (1.0, 0.5, 0.25)
_EPS = 0.001


def _pool_body(v_ref, s_ref, d_ref, o_ref, *, ratio):
    T, C = v_ref.shape[1], v_ref.shape[2]
    N = s_ref.shape[1]
    v = v_ref[0]

    c = v
    sh = 1
    while sh < T:
        c = c + jnp.concatenate(
            [jnp.zeros((sh, C), jnp.float32), c[: T - sh]], axis=0
        )
        sh *= 2

    start = s_ref[0, 0]
    dur = d_ref[0, 0]
    s = jnp.minimum(jnp.floor(start * ratio).astype(jnp.int32), T - 1)
    e = jnp.minimum(
        jnp.ceil((start + dur + _EPS) * ratio).astype(jnp.int32), T - 1
    )

    iota_t = lax.broadcasted_iota(jnp.int32, (N, T), 1)
    m = (iota_t == (e - 1)[:, None]).astype(jnp.float32) - (
        iota_t == (s - 1)[:, None]
    ).astype(jnp.float32)
    seg = lax.dot_general(
        m, c, (((1,), (0,)), ((), ())),
        preferred_element_type=jnp.float32,
        precision=lax.Precision.HIGHEST,
    )
    cnt = (e - s).astype(jnp.float32)
    mean = jnp.where(
        cnt[:, None] > 0, seg / jnp.maximum(cnt, 1.0)[:, None], 0.0
    )
    o_ref[0] = mean


def _pool_one(value, start, duration, ratio, *, interpret=False):
    B, T, C = value.shape
    N = start.shape[1]
    return pl.pallas_call(
        functools.partial(_pool_body, ratio=ratio),
        grid=(B,),
        in_specs=[
            pl.BlockSpec((1, T, C), lambda b: (b, 0, 0)),
            pl.BlockSpec((1, 1, N), lambda b: (b, 0, 0)),
            pl.BlockSpec((1, 1, N), lambda b: (b, 0, 0)),
        ],
        out_specs=pl.BlockSpec((1, N, C), lambda b: (b, 0, 0)),
        out_shape=jax.ShapeDtypeStruct((B, N, C), jnp.float32),
        interpret=interpret,
    )(value, start.reshape(B, 1, N), duration.reshape(B, 1, N))


def kernel(value_list_0, value_list_1, value_list_2, start, duration):
    vals = (value_list_0, value_list_1, value_list_2)
    pooled = [
        _pool_one(v, start, duration, r) for v, r in zip(vals, _RATIOS)
    ]
    return jnp.concatenate(pooled, axis=-1)
